# baseline (speedup 1.0000x reference)
import functools

import jax
import jax.numpy as jnp
from jax import lax
from jax.experimental import pallas as pl
from jax.experimental.pallas import tpu as pltpu
from jax.experimental.pallas import tpu_sc as plsc

N = 10000
B = 64
E = 320000
ITERS = 20
LEAK = 0.01

NC = 2
NS = 16
Bh = B // NC
R = N // NS
K = 128
NBUF = 2
JC = 2
EperT = -(-E // NS)
NCH = NBUF * JC * (-(-EperT // (NBUF * JC * K)))
NST = NCH // JC
EP = NS * NCH * K
RC = 125
NT = R // RC


def _act(v):
    return jnp.maximum(v, 0.0) + LEAK * jnp.minimum(v, 0.0)


def _pack(a, b):
    return plsc.pack(a, b, format=plsc.PackFormat.INTERLEAVED)


def _unpack(ab):
    return plsc.unpack(ab, format=plsc.PackFormat.INTERLEAVED)


def _sc_body(binc, crvp, wp, out, dlt_sh, dacc_sh, crv, wv, cidx, ridx,
             gbuf, sbuf, abuf, dbuf, dchunk, accl, xold,
             gsem0, gsem1, ssem0, ssem1, rsem, dsem, zsem):
    c = lax.axis_index("c")
    s = lax.axis_index("s")
    gsem = (gsem0, gsem1)
    ssem = (ssem0, ssem1)

    pltpu.sync_copy(crvp.at[s], crv)
    pltpu.sync_copy(wp.at[s], wv)

    pltpu.sync_copy(binc.at[c, s], accl)

    def zrow(r, carry):
        dchunk[0, r, pl.ds(0, 32)] = jnp.zeros((32,), jnp.bfloat16)
        dchunk[1, r, pl.ds(0, 32)] = jnp.zeros((32,), jnp.bfloat16)
        return carry

    lax.fori_loop(0, RC, zrow, 0)
    for t in range(NT):
        sl = pl.ds(s * R + t * RC, RC)

        def init_row(r, carry, t=t):
            v0 = _act(accl[t * RC + r, pl.ds(0, 16)])
            v1 = _act(accl[t * RC + r, pl.ds(16, 16)])
            dq = _pack(v0, v1)
            dbuf[t % 2, r, pl.ds(0, 32)] = dq
            e0, e1 = _unpack(dq)
            xold[t * RC + r, pl.ds(0, 16)] = e0
            xold[t * RC + r, pl.ds(16, 16)] = e1
            return carry

        lax.fori_loop(0, RC, init_row, 0)
        pltpu.sync_copy(dbuf.at[t % 2], dlt_sh.at[sl])
        pltpu.sync_copy(dchunk.at[t % 2], dacc_sh.at[sl])
    plsc.subcore_barrier()

    def scale(q2, b):
        for j in range(JC):
            for g in range(K // 16):
                wvec = wv[q2 * JC + j, pl.ds(g * 16, 16)]
                for k in range(16):
                    bc = jnp.take_along_axis(
                        wvec, jnp.full((16,), k, jnp.int32), axis=0)
                    r = g * 16 + k
                    v0, v1 = _unpack(gbuf[b, j, r, pl.ds(0, 32)])
                    sbuf[b, j, r, pl.ds(0, 32)] = _pack(v0 * bc, v1 * bc)

    def set_cidx(q2, b):
        for j in range(JC):
            for g in range(K // 16):
                v = crv[q2 * JC + j, pl.ds(g * 16, 16)]
                cidx[b, j, pl.ds(g * 16, 16)] = v & 0xFFFF

    def set_ridx(q2, b):
        for j in range(JC):
            for g in range(K // 16):
                v = crv[q2 * JC + j, pl.ds(g * 16, 16)]
                ridx[b, j, pl.ds(g * 16, 16)] = v >> 16

    def start_gather(b):
        for j in range(JC):
            pltpu.async_copy(dlt_sh.at[cidx.at[b, j]], gbuf.at[b, j],
                             gsem[b])

    def wait_gather(b):
        for j in range(JC):
            pltpu.make_async_copy(dlt_sh.at[cidx.at[b, j]], gbuf.at[b, j],
                                  gsem[b]).wait()

    def start_scatter(b):
        for j in range(JC):
            pltpu.async_copy(sbuf.at[b, j], dacc_sh.at[ridx.at[b, j]],
                             ssem[b], add=True)

    def wait_scatter(b):
        for j in range(JC):
            pltpu.make_async_copy(sbuf.at[b, j], dacc_sh.at[ridx.at[b, j]],
                                  ssem[b]).wait()

    def iteration(it, carry):
        for b in range(NBUF):
            set_cidx(b, b)
            start_gather(b)

        def rung(i, carry2):
            for b in range(NBUF):
                q2 = i * NBUF + b
                wait_gather(b)

                @pl.when(i > 0)
                def _():
                    wait_scatter(b)

                scale(q2, b)

                @pl.when(q2 + NBUF < NST)
                def _():
                    set_cidx(q2 + NBUF, b)
                    start_gather(b)

                set_ridx(q2, b)
                start_scatter(b)
            return carry2

        lax.fori_loop(0, NST // NBUF, rung, 0)
        for b in range(NBUF):
            wait_scatter(b)
        plsc.subcore_barrier()

        def rd_desc(t):
            sl = pl.ds(s * R + t * RC, RC)
            return (dacc_sh.at[sl], dchunk.at[t % 2], rsem)

        def dlt_desc(t):
            sl = pl.ds(s * R + t * RC, RC)
            return (dbuf.at[t % 2], dlt_sh.at[sl], dsem)

        def zero_desc(t):
            sl = pl.ds(s * R + t * RC, RC)
            return (dchunk.at[t % 2], dacc_sh.at[sl], zsem)

        pltpu.async_copy(*rd_desc(0))
        for t in range(NT):
            tb = t % 2
            pltpu.make_async_copy(*rd_desc(t)).wait()
            if t + 1 < NT:
                if t >= 1:
                    pltpu.make_async_copy(*zero_desc(t - 1)).wait()
                pltpu.async_copy(*rd_desc(t + 1))
            if t >= 2:
                pltpu.make_async_copy(*dlt_desc(t - 2)).wait()

            def upd_row(r, carry3, t=t, tb=tb):
                e0, e1 = _unpack(dchunk[tb, r, pl.ds(0, 32)])
                a0 = accl[t * RC + r, pl.ds(0, 16)] + e0
                a1 = accl[t * RC + r, pl.ds(16, 16)] + e1
                accl[t * RC + r, pl.ds(0, 16)] = a0
                accl[t * RC + r, pl.ds(16, 16)] = a1
                xn0 = _act(a0)
                xn1 = _act(a1)
                d0 = xn0 - xold[t * RC + r, pl.ds(0, 16)]
                d1 = xn1 - xold[t * RC + r, pl.ds(16, 16)]
                dq = _pack(d0, d1)
                dbuf[tb, r, pl.ds(0, 32)] = dq
                f0, f1 = _unpack(dq)
                xold[t * RC + r, pl.ds(0, 16)] += f0
                xold[t * RC + r, pl.ds(16, 16)] += f1
                abuf[r, pl.ds(0, 16)] = xn0
                abuf[r, pl.ds(16, 16)] = xn1
                dchunk[tb, r, pl.ds(0, 32)] = jnp.zeros((32,), jnp.bfloat16)
                return carry3

            lax.fori_loop(0, RC, upd_row, 0)
            pltpu.async_copy(*dlt_desc(t))
            pltpu.async_copy(*zero_desc(t))

            @pl.when(it == ITERS - 2)
            def _():
                pltpu.sync_copy(abuf, out.at[c, pl.ds(s * R + t * RC, RC)])

        for t in range(NT - 2, NT):
            pltpu.make_async_copy(*dlt_desc(t)).wait()
            pltpu.make_async_copy(*zero_desc(t)).wait()
        plsc.subcore_barrier()
        return carry

    lax.fori_loop(0, ITERS - 1, iteration, 0)


@jax.jit
def _run(binc, crvp, wp):
    f = pl.kernel(
        _sc_body,
        out_type=jax.ShapeDtypeStruct((NC, N, Bh), jnp.float32),
        mesh=plsc.VectorSubcoreMesh(core_axis_name="c", subcore_axis_name="s"),
        compiler_params=pltpu.CompilerParams(use_tc_tiling_on_sc=False,
                                             needs_layout_passes=False),
        scratch_types=[
            pltpu.VMEM_SHARED((N, Bh), jnp.bfloat16),
            pltpu.VMEM_SHARED((N, Bh), jnp.bfloat16),
            pltpu.VMEM((NCH, K), jnp.int32),
            pltpu.VMEM((NCH, K), jnp.float32),
            pltpu.VMEM((NBUF, JC, K), jnp.int32),
            pltpu.VMEM((NBUF, JC, K), jnp.int32),
            pltpu.VMEM((NBUF, JC, K, Bh), jnp.bfloat16),
            pltpu.VMEM((NBUF, JC, K, Bh), jnp.bfloat16),
            pltpu.VMEM((RC, Bh), jnp.float32),
            pltpu.VMEM((2, RC, Bh), jnp.bfloat16),
            pltpu.VMEM((2, RC, Bh), jnp.bfloat16),
            pltpu.VMEM((R, Bh), jnp.float32),
            pltpu.VMEM((R, Bh), jnp.float32),
            pltpu.SemaphoreType.DMA,
            pltpu.SemaphoreType.DMA,
            pltpu.SemaphoreType.DMA,
            pltpu.SemaphoreType.DMA,
            pltpu.SemaphoreType.DMA,
            pltpu.SemaphoreType.DMA,
            pltpu.SemaphoreType.DMA,
        ],
    )
    return f(binc, crvp, wp)


def kernel(x, weights, bias, row, col):
    row = row.astype(jnp.int32)
    col = col.astype(jnp.int32)
    weights = weights.astype(jnp.float32)
    pad = EP - E
    crv = (row << 16) | col
    crvp = jnp.concatenate([crv, jnp.zeros((pad,), jnp.int32)]).reshape(NS, NCH, K)
    wp = jnp.concatenate([weights, jnp.zeros((pad,), jnp.float32)]).reshape(NS, NCH, K)
    bIn = x.T + bias
    binc = bIn.reshape(N, NC, Bh).transpose(1, 0, 2)
    binc = binc.reshape(NC, NS, R, Bh)
    out = _run(binc, crvp, wp)
    return out.transpose(1, 0, 2).reshape(N, B).T


# speedup vs baseline: 1.6720x; 1.6720x over previous
import functools

import jax
import jax.numpy as jnp
from jax import lax
from jax.experimental import pallas as pl
from jax.experimental.pallas import tpu as pltpu
from jax.experimental.pallas import tpu_sc as plsc

N = 10000
B = 64
E = 320000
ITERS = 20
LEAK = 0.01

NC = 2
NS = 16
Bh = B // NC
R = N // NS
K = 128
NBUF = 2
EperT = -(-E // NS)
NCH = NBUF * (-(-EperT // (NBUF * K)))
EP = NS * NCH * K
RC = 125
NT = R // RC


def _act(v):
    return jnp.maximum(v, 0.0) + LEAK * jnp.minimum(v, 0.0)


def _pack(a, b):
    return plsc.pack(a, b, format=plsc.PackFormat.INTERLEAVED)


def _unpack(ab):
    return plsc.unpack(ab, format=plsc.PackFormat.INTERLEAVED)


def _sc_body(binc, crvp, wp, out, dlt_sh, dacc_sh, crv, wv, cidx, ridx,
             gbuf, sbuf, abuf, dbuf, dchunk, accl, xold,
             gsem0, gsem1, ssem0, ssem1, rsem, dsem, zsem):
    c = lax.axis_index("c")
    s = lax.axis_index("s")
    gsem = (gsem0, gsem1)
    ssem = (ssem0, ssem1)

    pltpu.sync_copy(crvp.at[s], crv)
    pltpu.sync_copy(wp.at[s], wv)

    pltpu.sync_copy(binc.at[c, s], accl)

    def zrow(r, carry):
        dchunk[0, r, pl.ds(0, 32)] = jnp.zeros((32,), jnp.bfloat16)
        dchunk[1, r, pl.ds(0, 32)] = jnp.zeros((32,), jnp.bfloat16)
        return carry

    lax.fori_loop(0, RC, zrow, 0)
    for t in range(NT):
        sl = pl.ds(s * R + t * RC, RC)

        def init_row(r, carry, t=t):
            v0 = _act(accl[t * RC + r, pl.ds(0, 16)])
            v1 = _act(accl[t * RC + r, pl.ds(16, 16)])
            dq = _pack(v0, v1)
            dbuf[t % 2, r, pl.ds(0, 32)] = dq
            e0, e1 = _unpack(dq)
            xold[t * RC + r, pl.ds(0, 16)] = e0
            xold[t * RC + r, pl.ds(16, 16)] = e1
            return carry

        lax.fori_loop(0, RC, init_row, 0)
        pltpu.sync_copy(dbuf.at[t % 2], dlt_sh.at[sl])
        pltpu.sync_copy(dchunk.at[t % 2], dacc_sh.at[sl])
    plsc.subcore_barrier()

    def scale(q, b):
        for g in range(K // 16):
            wvec = wv[q, pl.ds(g * 16, 16)]
            for k in range(16):
                bc = jnp.take_along_axis(
                    wvec, jnp.full((16,), k, jnp.int32), axis=0)
                r = g * 16 + k
                v0, v1 = _unpack(gbuf[b, r, pl.ds(0, 32)])
                sbuf[b, r, pl.ds(0, 32)] = _pack(v0 * bc, v1 * bc)

    def set_cidx(q, b):
        for g in range(K // 16):
            v = crv[q, pl.ds(g * 16, 16)]
            cidx[b, pl.ds(g * 16, 16)] = v & 0xFFFF

    def set_ridx(q, b):
        for g in range(K // 16):
            v = crv[q, pl.ds(g * 16, 16)]
            ridx[b, pl.ds(g * 16, 16)] = v >> 16

    def start_gather(b):
        return pltpu.async_copy(dlt_sh.at[cidx.at[b]], gbuf.at[b], gsem[b])

    def wait_gather(b):
        pltpu.make_async_copy(dlt_sh.at[cidx.at[b]], gbuf.at[b],
                              gsem[b]).wait()

    def start_scatter(b):
        return pltpu.async_copy(sbuf.at[b], dacc_sh.at[ridx.at[b]],
                                ssem[b], add=True)

    def wait_scatter(b):
        pltpu.make_async_copy(sbuf.at[b], dacc_sh.at[ridx.at[b]],
                              ssem[b]).wait()

    def iteration(it, carry):
        for b in range(NBUF):
            set_cidx(b, b)
            start_gather(b)

        def rung(i, carry2):
            for b in range(NBUF):
                q = i * NBUF + b
                wait_gather(b)

                @pl.when(i > 0)
                def _():
                    wait_scatter(b)

                scale(q, b)

                @pl.when(q + NBUF < NCH)
                def _():
                    set_cidx(q + NBUF, b)
                    start_gather(b)

                set_ridx(q, b)
                start_scatter(b)
            return carry2

        lax.fori_loop(0, NCH // NBUF, rung, 0)
        for b in range(NBUF):
            wait_scatter(b)
        plsc.subcore_barrier()

        def rd_desc(t):
            sl = pl.ds(s * R + t * RC, RC)
            return (dacc_sh.at[sl], dchunk.at[t % 2], rsem)

        def dlt_desc(t):
            sl = pl.ds(s * R + t * RC, RC)
            return (dbuf.at[t % 2], dlt_sh.at[sl], dsem)

        def zero_desc(t):
            sl = pl.ds(s * R + t * RC, RC)
            return (dchunk.at[t % 2], dacc_sh.at[sl], zsem)

        pltpu.async_copy(*rd_desc(0))
        for t in range(NT):
            tb = t % 2
            pltpu.make_async_copy(*rd_desc(t)).wait()
            if t + 1 < NT:
                if t >= 1:
                    pltpu.make_async_copy(*zero_desc(t - 1)).wait()
                pltpu.async_copy(*rd_desc(t + 1))
            if t >= 2:
                pltpu.make_async_copy(*dlt_desc(t - 2)).wait()

            def upd_row(r, carry3, t=t, tb=tb):
                e0, e1 = _unpack(dchunk[tb, r, pl.ds(0, 32)])
                a0 = accl[t * RC + r, pl.ds(0, 16)] + e0
                a1 = accl[t * RC + r, pl.ds(16, 16)] + e1
                accl[t * RC + r, pl.ds(0, 16)] = a0
                accl[t * RC + r, pl.ds(16, 16)] = a1
                xn0 = _act(a0)
                xn1 = _act(a1)
                d0 = xn0 - xold[t * RC + r, pl.ds(0, 16)]
                d1 = xn1 - xold[t * RC + r, pl.ds(16, 16)]
                dq = _pack(d0, d1)
                dbuf[tb, r, pl.ds(0, 32)] = dq
                f0, f1 = _unpack(dq)
                xold[t * RC + r, pl.ds(0, 16)] += f0
                xold[t * RC + r, pl.ds(16, 16)] += f1
                abuf[r, pl.ds(0, 16)] = xn0
                abuf[r, pl.ds(16, 16)] = xn1
                dchunk[tb, r, pl.ds(0, 32)] = jnp.zeros((32,), jnp.bfloat16)
                return carry3

            lax.fori_loop(0, RC, upd_row, 0)
            pltpu.async_copy(*dlt_desc(t))
            pltpu.async_copy(*zero_desc(t))

            @pl.when(it == ITERS - 2)
            def _():
                pltpu.sync_copy(abuf, out.at[c, pl.ds(s * R + t * RC, RC)])

        for t in range(NT - 2, NT):
            pltpu.make_async_copy(*dlt_desc(t)).wait()
            pltpu.make_async_copy(*zero_desc(t)).wait()
        plsc.subcore_barrier()
        return carry

    lax.fori_loop(0, ITERS - 1, iteration, 0)


@jax.jit
def _run(binc, crvp, wp):
    f = pl.kernel(
        _sc_body,
        out_type=jax.ShapeDtypeStruct((NC, N, Bh), jnp.float32),
        mesh=plsc.VectorSubcoreMesh(core_axis_name="c", subcore_axis_name="s"),
        compiler_params=pltpu.CompilerParams(use_tc_tiling_on_sc=False,
                                             needs_layout_passes=False),
        scratch_types=[
            pltpu.VMEM_SHARED((N, Bh), jnp.bfloat16),
            pltpu.VMEM_SHARED((N, Bh), jnp.bfloat16),
            pltpu.VMEM((NCH, K), jnp.int32),
            pltpu.VMEM((NCH, K), jnp.float32),
            pltpu.VMEM((NBUF, K), jnp.int32),
            pltpu.VMEM((NBUF, K), jnp.int32),
            pltpu.VMEM((NBUF, K, Bh), jnp.bfloat16),
            pltpu.VMEM((NBUF, K, Bh), jnp.bfloat16),
            pltpu.VMEM((RC, Bh), jnp.float32),
            pltpu.VMEM((2, RC, Bh), jnp.bfloat16),
            pltpu.VMEM((2, RC, Bh), jnp.bfloat16),
            pltpu.VMEM((R, Bh), jnp.float32),
            pltpu.VMEM((R, Bh), jnp.float32),
            pltpu.SemaphoreType.DMA,
            pltpu.SemaphoreType.DMA,
            pltpu.SemaphoreType.DMA,
            pltpu.SemaphoreType.DMA,
            pltpu.SemaphoreType.DMA,
            pltpu.SemaphoreType.DMA,
            pltpu.SemaphoreType.DMA,
        ],
    )
    return f(binc, crvp, wp)


def kernel(x, weights, bias, row, col):
    row = row.astype(jnp.int32)
    col = col.astype(jnp.int32)
    weights = weights.astype(jnp.float32)
    pad = EP - E
    crv = (row << 16) | col
    crvp = jnp.concatenate([crv, jnp.zeros((pad,), jnp.int32)]).reshape(NS, NCH, K)
    wp = jnp.concatenate([weights, jnp.zeros((pad,), jnp.float32)]).reshape(NS, NCH, K)
    bIn = x.T + bias
    binc = bIn.reshape(N, NC, Bh).transpose(1, 0, 2)
    binc = binc.reshape(NC, NS, R, Bh)
    out = _run(binc, crvp, wp)
    return out.transpose(1, 0, 2).reshape(N, B).T
